# Initial kernel scaffold; baseline (speedup 1.0000x reference)
import jax
import jax.numpy as jnp
from jax.experimental import pallas as pl


def kernel(node_values, node_indices, roots):
    raise NotImplementedError("write your pallas kernel here")


# speedup vs baseline: 5.9185x; 5.9185x over previous
import functools

import jax
import jax.numpy as jnp
from jax import lax
from jax.experimental import pallas as pl
from jax.experimental.pallas import tpu as pltpu
from jax.experimental.pallas import tpu_sc as plsc

N = 32768
D = 256
B = 16
NC = 2
NS = 16
L = 16
NW = NC * NS
HALF = D // 2
CH = HALF // L
BLK = 256
NBLK = N // BLK

_mesh = plsc.VectorSubcoreMesh(
    core_axis_name="c", subcore_axis_name="s", num_cores=NC, num_subcores=NS
)


@functools.partial(
    pl.kernel,
    out_type=jax.ShapeDtypeStruct((B, 3 * D), jnp.float32),
    mesh=_mesh,
    scratch_types=[
        pltpu.VMEM((N + L,), jnp.int32),
        pltpu.VMEM((BLK, HALF), jnp.float32),
        pltpu.VMEM((2, HALF), jnp.float32),
        pltpu.VMEM((B,), jnp.int32),
        pltpu.VMEM((B, D), jnp.float32),
        pltpu.SemaphoreType.DMA,
    ],
)
def _state_embedding_kernel(values_hbm, idx_hbm, roots_hbm, out_hbm,
                            idx_v, rows_v, obuf, roots_v, rootrows_v, sem):
    wid = lax.axis_index("s") * NC + lax.axis_index("c")
    j = wid >> 1
    h = wid & 1
    col = h * HALF
    zeros = jnp.zeros((L,), jnp.float32)
    ninf = jnp.full((L,), -jnp.inf, jnp.float32)

    pltpu.sync_copy(idx_hbm, idx_v.at[pl.ds(0, N)])
    idx_v[pl.ds(N, L)] = jnp.zeros((L,), jnp.int32) + B

    def lower_bound(tgt):
        def step(_, carry):
            lo, hi = carry
            mid = (lo + hi) >> 1
            v = idx_v[pl.ds(mid, L)]
            lt = v[0] < tgt
            return jnp.where(lt, mid + 1, lo), jnp.where(lt, hi, mid)

        lo, _ = lax.fori_loop(0, 16, step, (jnp.int32(0), jnp.int32(N)))
        return lo

    lo_j = lower_bound(j)
    hi_j = lower_bound(j + 1)

    b_lo = lo_j >> 8
    b_hi = (hi_j + (BLK - 1)) >> 8

    def blk_body(b, carry):
        pltpu.sync_copy(
            values_hbm.at[pl.ds(b * BLK, BLK), pl.ds(col, HALF)], rows_v)

        def row_body(r, car):
            sums, maxs = car
            abs_r = b * BLK + r
            inr = jnp.logical_and(abs_r >= lo_j, abs_r < hi_j)
            pf = jnp.where(inr, 1.0, 0.0) + zeros
            pm = jnp.where(inr, 0.0, -jnp.inf) + zeros
            new_s = []
            new_m = []
            for c in range(CH):
                v = rows_v[r, pl.ds(c * L, L)]
                new_s.append(sums[c] + v * pf)
                new_m.append(jnp.maximum(maxs[c], v + pm))
            return tuple(new_s), tuple(new_m)

        return lax.fori_loop(0, BLK, row_body, carry)

    init = (tuple(zeros for _ in range(CH)), tuple(ninf for _ in range(CH)))
    sums, maxs = lax.fori_loop(b_lo, b_hi, blk_body, init)

    cntf = (hi_j - lo_j).astype(jnp.float32)
    rden = 1.0 / jnp.maximum(zeros + cntf, 1.0)
    for c in range(CH):
        obuf[0, pl.ds(c * L, L)] = sums[c] * rden
        obuf[1, pl.ds(c * L, L)] = maxs[c]
    pltpu.sync_copy(obuf.at[0], out_hbm.at[j, pl.ds(col, HALF)])
    pltpu.sync_copy(obuf.at[1], out_hbm.at[j, pl.ds(D + col, HALF)])

    @pl.when(wid == 0)
    def _():
        pltpu.sync_copy(roots_hbm, roots_v)
        pltpu.async_copy(values_hbm.at[roots_v], rootrows_v, sem).wait()
        pltpu.sync_copy(rootrows_v, out_hbm.at[:, pl.ds(2 * D, D)])


def kernel(node_values, node_indices, roots):
    idx32 = node_indices.astype(jnp.int32)
    roots32 = roots.astype(jnp.int32)
    return _state_embedding_kernel(node_values, idx32, roots32)


# speedup vs baseline: 7.9067x; 1.3359x over previous
import functools

import jax
import jax.numpy as jnp
from jax import lax
from jax.experimental import pallas as pl
from jax.experimental.pallas import tpu as pltpu
from jax.experimental.pallas import tpu_sc as plsc

N = 32768
D = 256
B = 16
NC = 2
NS = 16
L = 16
NW = NC * NS
HALF = D // 2
CH = HALF // L
BLK = 256
NBLK = N // BLK

_mesh = plsc.VectorSubcoreMesh(
    core_axis_name="c", subcore_axis_name="s", num_cores=NC, num_subcores=NS
)


@functools.partial(
    pl.kernel,
    out_type=jax.ShapeDtypeStruct((B, 3 * D), jnp.float32),
    mesh=_mesh,
    scratch_types=[
        pltpu.VMEM((N + L,), jnp.int32),
        pltpu.VMEM((2 * BLK, HALF), jnp.float32),
        pltpu.VMEM((2, HALF), jnp.float32),
        pltpu.VMEM((B,), jnp.int32),
        pltpu.VMEM((B, D), jnp.float32),
        pltpu.SemaphoreType.DMA,
        pltpu.SemaphoreType.DMA,
        pltpu.SemaphoreType.DMA,
    ],
)
def _state_embedding_kernel(values_hbm, idx_hbm, roots_hbm, out_hbm,
                            idx_v, rows_v, obuf, roots_v, rootrows_v,
                            sem0, sem1, rsem):
    wid = lax.axis_index("s") * NC + lax.axis_index("c")
    j = wid >> 1
    h = wid & 1
    col = h * HALF
    zeros = jnp.zeros((L,), jnp.float32)
    ninf = jnp.full((L,), -jnp.inf, jnp.float32)

    @pl.when(wid == 0)
    def _():
        pltpu.sync_copy(roots_hbm, roots_v)
        pltpu.async_copy(values_hbm.at[roots_v], rootrows_v, rsem)

    pltpu.sync_copy(idx_hbm, idx_v.at[pl.ds(0, N)])
    idx_v[pl.ds(N, L)] = jnp.zeros((L,), jnp.int32) + B

    def lower_bound(tgt):
        def step(_, carry):
            lo, hi = carry
            mid = (lo + hi) >> 1
            v = idx_v[pl.ds(mid, L)]
            lt = v[0] < tgt
            return jnp.where(lt, mid + 1, lo), jnp.where(lt, hi, mid)

        lo, _ = lax.fori_loop(0, 16, step, (jnp.int32(0), jnp.int32(N)))
        return lo

    lo_j = lower_bound(j)
    hi_j = lower_bound(j + 1)

    b_lo = lo_j >> 8
    b_hi = (hi_j + (BLK - 1)) >> 8
    nb = b_hi - b_lo

    def src(i):
        return values_hbm.at[pl.ds((b_lo + i) * BLK, BLK), pl.ds(col, HALF)]

    def start(i):
        @pl.when((i & 1) == 0)
        def _():
            pltpu.async_copy(src(i), rows_v.at[pl.ds(0, BLK)], sem0)

        @pl.when((i & 1) == 1)
        def _():
            pltpu.async_copy(src(i), rows_v.at[pl.ds(BLK, BLK)], sem1)

    def wait(i):
        @pl.when((i & 1) == 0)
        def _():
            pltpu.make_async_copy(src(i), rows_v.at[pl.ds(0, BLK)], sem0).wait()

        @pl.when((i & 1) == 1)
        def _():
            pltpu.make_async_copy(src(i), rows_v.at[pl.ds(BLK, BLK)], sem1).wait()

    @pl.when(nb > 0)
    def _():
        start(jnp.int32(0))

    def make_blk_body(masked):
        def blk_body(i, carry):
            @pl.when(i + 1 < nb)
            def _():
                start(i + 1)

            wait(i)
            b = b_lo + i
            base = (i & 1) * BLK

            def row_body(r, car):
                sums, maxs = car
                new_s = []
                new_m = []
                if masked:
                    abs_r = b * BLK + r
                    inr = jnp.logical_and(abs_r >= lo_j, abs_r < hi_j)
                    pf = jnp.where(inr, 1.0, 0.0) + zeros
                    pm = jnp.where(inr, 0.0, -jnp.inf) + zeros
                    for c in range(CH):
                        v = rows_v[base + r, pl.ds(c * L, L)]
                        new_s.append(sums[c] + v * pf)
                        new_m.append(jnp.maximum(maxs[c], v + pm))
                else:
                    for c in range(CH):
                        v = rows_v[base + r, pl.ds(c * L, L)]
                        new_s.append(sums[c] + v)
                        new_m.append(jnp.maximum(maxs[c], v))
                return tuple(new_s), tuple(new_m)

            return lax.fori_loop(0, BLK, row_body, carry)

        return blk_body

    edge_body = make_blk_body(True)
    full_body = make_blk_body(False)

    init = (tuple(zeros for _ in range(CH)), tuple(ninf for _ in range(CH)))
    carry = lax.fori_loop(0, jnp.minimum(nb, 1), edge_body, init)
    carry = lax.fori_loop(1, nb - 1, full_body, carry)
    sums, maxs = lax.fori_loop(jnp.maximum(nb - 1, 1), nb, edge_body, carry)

    cntf = (hi_j - lo_j).astype(jnp.float32)
    rden = 1.0 / jnp.maximum(zeros + cntf, 1.0)
    for c in range(CH):
        obuf[0, pl.ds(c * L, L)] = sums[c] * rden
        obuf[1, pl.ds(c * L, L)] = maxs[c]
    pltpu.sync_copy(obuf.at[0], out_hbm.at[j, pl.ds(col, HALF)])
    pltpu.sync_copy(obuf.at[1], out_hbm.at[j, pl.ds(D + col, HALF)])

    @pl.when(wid == 0)
    def _():
        pltpu.make_async_copy(values_hbm.at[roots_v], rootrows_v, rsem).wait()
        pltpu.sync_copy(rootrows_v, out_hbm.at[:, pl.ds(2 * D, D)])


def kernel(node_values, node_indices, roots):
    idx32 = node_indices.astype(jnp.int32)
    roots32 = roots.astype(jnp.int32)
    return _state_embedding_kernel(node_values, idx32, roots32)
